# baseline (speedup 1.0000x reference)
import functools

import jax
import jax.numpy as jnp
from jax import lax
from jax.experimental import pallas as pl
from jax.experimental.pallas import tpu as pltpu
from jax.experimental.pallas import tpu_sc as plsc

N_CODES = 1024
D = 64
BETA = 0.25
TILE = 512

_NC, _NS = 2, 16
_NW = _NC * _NS
_IDX_CHUNK = 128


def _vq_body(z_ref, cbm2_ref, e2_ref, idx_ref, mr_ref):
    zb = z_ref[0]
    zt = zb.T
    zem2 = jax.lax.dot_general(zt, cbm2_ref[...], (((1,), (1,)), ((), ())),
                               preferred_element_type=jnp.float32)
    z2 = jnp.sum(zt * zt, axis=1, keepdims=True)
    dists = (z2 + e2_ref[...]) + zem2
    m = jnp.min(dists, axis=1, keepdims=True)
    iota_f = jax.lax.broadcasted_iota(
        jnp.int32, (TILE, N_CODES), 1).astype(jnp.float32)
    idx_f = jnp.min(jnp.where(dists == m, iota_f, float(N_CODES)), axis=1)
    idx_ref[0, 0, 0] = idx_f.astype(jnp.int32)
    mr_ref[0, 0, 0] = m[:, 0]


_PAD_D = 128
_ROW_BATCH = 512


def _sc_gather_body(table_hbm, idx_hbm, out_hbm, idx_v, rows_v, sem):
    wid = lax.axis_index("s") * _NC + lax.axis_index("c")
    n_per_w = idx_v.shape[0]
    base = wid * n_per_w
    pltpu.sync_copy(idx_hbm.at[pl.ds(base, n_per_w)], idx_v)
    for r0 in range(0, n_per_w, _ROW_BATCH):
        copies = []
        for j in range(r0, r0 + _ROW_BATCH, _IDX_CHUNK):
            copies.append(pltpu.async_copy(
                table_hbm.at[idx_v.at[pl.ds(j, _IDX_CHUNK)]],
                rows_v.at[pl.ds(j - r0, _IDX_CHUNK)], sem))
        for c in copies:
            c.wait()
        pltpu.sync_copy(rows_v, out_hbm.at[pl.ds(base + r0, _ROW_BATCH)])


def _sc_gather(table_pad, idx_flat):
    n = idx_flat.shape[0]
    n_per_w = n // _NW
    mesh = plsc.VectorSubcoreMesh(core_axis_name="c", subcore_axis_name="s")
    k = functools.partial(
        pl.kernel, mesh=mesh,
        out_type=jax.ShapeDtypeStruct((n, _PAD_D), jnp.float32),
        scratch_types=[
            pltpu.VMEM((n_per_w,), jnp.int32),
            pltpu.VMEM((_ROW_BATCH, _PAD_D), jnp.float32),
            pltpu.SemaphoreType.DMA,
        ],
    )(_sc_gather_body)
    return k(table_pad, idx_flat)


def kernel(z, codebook):
    B, Dc, T, H, W = z.shape
    S = T * H * W
    n_chunks = S // TILE
    z3 = z.reshape(B, Dc, S)
    cbm2 = -2.0 * codebook
    e2 = jnp.sum(codebook ** 2, axis=1)[None, :]
    grid = (B, n_chunks)
    idx4, mrow = pl.pallas_call(
        _vq_body,
        grid=grid,
        in_specs=[
            pl.BlockSpec((1, Dc, TILE), lambda b, c: (b, 0, c)),
            pl.BlockSpec((N_CODES, Dc), lambda b, c: (0, 0)),
            pl.BlockSpec((1, N_CODES), lambda b, c: (0, 0)),
        ],
        out_specs=[
            pl.BlockSpec((1, 1, 1, TILE), lambda b, c: (b, c, 0, 0)),
            pl.BlockSpec((1, 1, 1, TILE), lambda b, c: (b, c, 0, 0)),
        ],
        out_shape=[
            jax.ShapeDtypeStruct((B, n_chunks, 1, TILE), jnp.int32),
            jax.ShapeDtypeStruct((B, n_chunks, 1, TILE), jnp.float32),
        ],
    )(z3, cbm2, e2)
    idx_flat = idx4.reshape(-1)
    cb_pad = jnp.pad(codebook, ((0, 0), (0, _PAD_D - Dc)))
    zq_rows = _sc_gather(cb_pad, idx_flat)[:, :Dc]
    z_q = jnp.transpose(zq_rows.reshape(B, T, H, W, Dc), (0, 4, 1, 2, 3))
    z_q_st = z + (z_q - z)
    idx = idx4.reshape(B, T, H, W)
    v = jnp.sum(mrow) / (B * S * Dc)
    vq_loss = v + BETA * v
    return z_q_st, vq_loss, idx


# speedup vs baseline: 1.1480x; 1.1480x over previous
import functools

import jax
import jax.numpy as jnp
from jax import lax
from jax.experimental import pallas as pl
from jax.experimental.pallas import tpu as pltpu
from jax.experimental.pallas import tpu_sc as plsc

N_CODES = 1024
D = 64
BETA = 0.25
TILE = 512

_NC, _NS = 2, 16
_NW = _NC * _NS
_IDX_CHUNK = 128


def _vq_body(z_ref, cbm2_ref, e2_ref, idx_ref):
    zb = z_ref[0]
    zt = zb.T
    zem2 = jax.lax.dot_general(zt, cbm2_ref[...], (((1,), (1,)), ((), ())),
                               preferred_element_type=jnp.float32)
    z2 = jnp.sum(zt * zt, axis=1, keepdims=True)
    dists = (z2 + e2_ref[...]) + zem2
    m = jnp.min(dists, axis=1, keepdims=True)
    iota_f = jax.lax.broadcasted_iota(
        jnp.int32, (TILE, N_CODES), 1).astype(jnp.float32)
    idx_f = jnp.min(jnp.where(dists == m, iota_f, float(N_CODES)), axis=1)
    idx_ref[0, 0, 0] = idx_f.astype(jnp.int32)


_PAD_D = 128
_ROW_BATCH = 512


def _sc_gather_body(table_hbm, idx_hbm, out_hbm, idx_v, rows_v, sem):
    wid = lax.axis_index("s") * _NC + lax.axis_index("c")
    n_per_w = idx_v.shape[0]
    base = wid * n_per_w
    pltpu.sync_copy(idx_hbm.at[pl.ds(base, n_per_w)], idx_v)
    for r0 in range(0, n_per_w, _ROW_BATCH):
        copies = []
        for j in range(r0, r0 + _ROW_BATCH, _IDX_CHUNK):
            copies.append(pltpu.async_copy(
                table_hbm.at[idx_v.at[pl.ds(j, _IDX_CHUNK)]],
                rows_v.at[pl.ds(j - r0, _IDX_CHUNK)], sem))
        for c in copies:
            c.wait()
        pltpu.sync_copy(rows_v, out_hbm.at[pl.ds(base + r0, _ROW_BATCH)])


def _sc_gather(table_pad, idx_flat):
    n = idx_flat.shape[0]
    n_per_w = n // _NW
    mesh = plsc.VectorSubcoreMesh(core_axis_name="c", subcore_axis_name="s")
    k = functools.partial(
        pl.kernel, mesh=mesh,
        out_type=jax.ShapeDtypeStruct((n, _PAD_D), jnp.float32),
        scratch_types=[
            pltpu.VMEM((n_per_w,), jnp.int32),
            pltpu.VMEM((_ROW_BATCH, _PAD_D), jnp.float32),
            pltpu.SemaphoreType.DMA,
        ],
    )(_sc_gather_body)
    return k(table_pad, idx_flat)


def kernel(z, codebook):
    B, Dc, T, H, W = z.shape
    S = T * H * W
    n_chunks = S // TILE
    z3 = z.reshape(B, Dc, S)
    cbm2 = -2.0 * codebook
    e2 = jnp.sum(codebook ** 2, axis=1)[None, :]
    grid = (B, n_chunks)
    idx4 = pl.pallas_call(
        _vq_body,
        grid=grid,
        in_specs=[
            pl.BlockSpec((1, Dc, TILE), lambda b, c: (b, 0, c)),
            pl.BlockSpec((N_CODES, Dc), lambda b, c: (0, 0)),
            pl.BlockSpec((1, N_CODES), lambda b, c: (0, 0)),
        ],
        out_specs=pl.BlockSpec((1, 1, 1, TILE), lambda b, c: (b, c, 0, 0)),
        out_shape=jax.ShapeDtypeStruct((B, n_chunks, 1, TILE), jnp.int32),
    )(z3, cbm2, e2)
    idx_flat = idx4.reshape(-1)
    cb_pad = jnp.pad(codebook, ((0, 0), (0, _PAD_D - Dc)))
    zq_rows = _sc_gather(cb_pad, idx_flat)[:, :Dc]
    z_q = jnp.transpose(zq_rows.reshape(B, T, H, W, Dc), (0, 4, 1, 2, 3))
    z_q_st = z + (z_q - z)
    idx = idx4.reshape(B, T, H, W)
    v = jnp.mean((z_q - z) ** 2)
    vq_loss = v + BETA * v
    return z_q_st, vq_loss, idx
